# baseline (device time: 121271 ns/iter reference)
import jax
import jax.numpy as jnp
from jax import lax
from jax.experimental import pallas as pl
from jax.experimental.pallas import tpu as pltpu

N_DEV = 4
SQ = 2048
D_MODEL = 1024
HQ = 8
DH = 128
QB = 256
WIN = 128
KW = 512
SCALE = 0.08838834764831843
CHUNK = SQ // N_DEV
HALF = D_MODEL // 2
NSTEPS = 2 * (N_DEV - 1)

BF = jnp.bfloat16


def _body(x_ref, wq_ref, k_ref, v_ref, wo_ref, out_ref,
          part_ref, kbf_ref, vbf_ref, ctx_ref,
          comm_r, comm_l, send_r, recv_r, send_l, recv_l):
    my = lax.axis_index("i")
    left = (my - 1) % N_DEV
    right = (my + 1) % N_DEV

    barrier_sem = pltpu.get_barrier_semaphore()
    for nbr in (left, right):
        pl.semaphore_signal(
            barrier_sem, inc=1,
            device_id=(nbr,), device_id_type=pl.DeviceIdType.MESH,
        )
    pl.semaphore_wait(barrier_sem, 2)

    for h in range(HQ):
        kbf_ref[h, :, :] = k_ref[:, h, :].astype(BF)
        vbf_ref[h, :, :] = v_ref[:, h, :].astype(BF)

    def compute_chunk(c):
        for sub in range(CHUNK // QB):
            q0 = pl.multiple_of(c * CHUNK + sub * QB, QB)
            lo = pl.multiple_of(jnp.clip(q0 - WIN, 0, SQ - KW), WIN)
            x_blk = x_ref[pl.ds(q0, QB), :].astype(BF)
            q_all = (jnp.dot(x_blk, wq_ref[:, :],
                             preferred_element_type=jnp.float32)
                     * SCALE).astype(BF)
            for h in range(HQ):
                q_h = q_all[:, h * DH:(h + 1) * DH]
                k_h = kbf_ref[h, pl.ds(lo, KW), :]
                v_h = vbf_ref[h, pl.ds(lo, KW), :]
                s = lax.dot_general(q_h, k_h, (((1,), (1,)), ((), ())),
                                    preferred_element_type=jnp.float32)
                qi = q0 + lax.broadcasted_iota(jnp.int32, (QB, KW), 0)
                ki = lo + lax.broadcasted_iota(jnp.int32, (QB, KW), 1)
                s = jnp.where(jnp.abs(qi - ki) <= WIN, s, -1e9)
                m = jnp.max(s, axis=1, keepdims=True)
                w = jnp.exp(s - m)
                w = (w / jnp.sum(w, axis=1, keepdims=True)).astype(BF)
                ctx_ref[:, h * DH:(h + 1) * DH] = jnp.dot(
                    w, v_h, preferred_element_type=jnp.float32).astype(BF)
            part_ref[pl.ds(q0, QB), :] = jnp.dot(
                ctx_ref[:, :], wo_ref[:, :],
                preferred_element_type=jnp.float32).astype(BF)

    def rows_r(c):
        return part_ref.at[pl.ds(c * CHUNK, CHUNK), pl.ds(0, HALF)]

    def rows_l(c):
        return part_ref.at[pl.ds(c * CHUNK, CHUNK), pl.ds(HALF, HALF)]

    def acc_r(c, slot):
        part_ref[pl.ds(c * CHUNK, CHUNK), pl.ds(0, HALF)] = (
            part_ref[pl.ds(c * CHUNK, CHUNK), pl.ds(0, HALF)]
            + comm_r[slot])

    def acc_l(c, slot):
        part_ref[pl.ds(c * CHUNK, CHUNK), pl.ds(HALF, HALF)] = (
            part_ref[pl.ds(c * CHUNK, CHUNK), pl.ds(HALF, HALF)]
            + comm_l[slot])

    def rdma_r(src, slot, dev):
        return pltpu.make_async_remote_copy(
            src_ref=src, dst_ref=comm_r.at[slot],
            send_sem=send_r.at[slot], recv_sem=recv_r.at[slot],
            device_id=(dev,), device_id_type=pl.DeviceIdType.MESH)

    def rdma_l(src, slot, dev):
        return pltpu.make_async_remote_copy(
            src_ref=src, dst_ref=comm_l.at[slot],
            send_sem=send_l.at[slot], recv_sem=recv_l.at[slot],
            device_id=(dev,), device_id_type=pl.DeviceIdType.MESH)

    c0 = my
    c1 = (my - 1) % N_DEV
    c2 = (my + 1) % N_DEV
    c3 = (my + 2) % N_DEV

    compute_chunk(c0)
    r0 = rdma_r(rows_r(c0), 0, right); r0.start()
    l0 = rdma_l(rows_l(c0), 0, left); l0.start()

    compute_chunk(c1)
    r0.wait_recv()
    acc_r(c1, 0)
    r1 = rdma_r(rows_r(c1), 1, right); r1.start()

    compute_chunk(c2)
    l0.wait_recv()
    acc_l(c2, 0)
    l1 = rdma_l(rows_l(c2), 1, left); l1.start()

    compute_chunk(c3)
    r1.wait_recv()
    acc_r(c3, 1)
    r2 = rdma_r(rows_r(c3), 2, right); r2.start()
    l1.wait_recv()
    acc_l(c3, 1)
    l2 = rdma_l(rows_l(c3), 2, left); l2.start()

    r2.wait_recv()
    acc_r(c2, 2)
    ar0 = rdma_r(rows_r(c2), 3, right); ar0.start()
    l2.wait_recv()
    acc_l(c1, 2)
    al0 = rdma_l(rows_l(c1), 3, left); al0.start()

    out_ref[pl.ds(c2 * CHUNK, CHUNK), pl.ds(0, HALF)] = part_ref[
        pl.ds(c2 * CHUNK, CHUNK), pl.ds(0, HALF)].astype(jnp.float32)
    out_ref[pl.ds(c1 * CHUNK, CHUNK), pl.ds(HALF, HALF)] = part_ref[
        pl.ds(c1 * CHUNK, CHUNK), pl.ds(HALF, HALF)].astype(jnp.float32)

    ar0.wait_recv()
    ar1 = rdma_r(comm_r.at[3], 4, right); ar1.start()
    out_ref[pl.ds(c0 * CHUNK, CHUNK), pl.ds(0, HALF)] = (
        comm_r[3].astype(jnp.float32))
    al0.wait_recv()
    al1 = rdma_l(comm_l.at[3], 4, left); al1.start()
    out_ref[pl.ds(c0 * CHUNK, CHUNK), pl.ds(HALF, HALF)] = (
        comm_l[3].astype(jnp.float32))

    ar1.wait_recv()
    ar2 = rdma_r(comm_r.at[4], 5, right); ar2.start()
    out_ref[pl.ds(c1 * CHUNK, CHUNK), pl.ds(0, HALF)] = (
        comm_r[4].astype(jnp.float32))
    al1.wait_recv()
    al2 = rdma_l(comm_l.at[4], 5, left); al2.start()
    out_ref[pl.ds(c2 * CHUNK, CHUNK), pl.ds(HALF, HALF)] = (
        comm_l[4].astype(jnp.float32))

    ar2.wait_recv()
    out_ref[pl.ds(c3 * CHUNK, CHUNK), pl.ds(0, HALF)] = (
        comm_r[5].astype(jnp.float32))
    al2.wait_recv()
    out_ref[pl.ds(c3 * CHUNK, CHUNK), pl.ds(HALF, HALF)] = (
        comm_l[5].astype(jnp.float32))

    for r in (r0, r1, r2, l0, l1, l2, ar0, ar1, ar2, al0, al1, al2):
        r.wait_send()


def kernel(x, Wq, K_ext, V_ext, Wo):
    my = lax.axis_index("i")
    d_loc = HQ * DH
    wq_loc = lax.dynamic_slice(
        Wq, (0, my * d_loc), (Wq.shape[0], d_loc)).astype(BF)
    wo_loc = lax.dynamic_slice(
        Wo, (my * d_loc, 0), (d_loc, Wo.shape[1])).astype(BF)

    out = pl.pallas_call(
        _body,
        out_shape=jax.ShapeDtypeStruct((SQ, D_MODEL), jnp.float32),
        in_specs=[pl.BlockSpec(memory_space=pltpu.VMEM)] * 5,
        out_specs=pl.BlockSpec(memory_space=pltpu.VMEM),
        scratch_shapes=[
            pltpu.VMEM((SQ, D_MODEL), BF),
            pltpu.VMEM((HQ, SQ, DH), BF),
            pltpu.VMEM((HQ, SQ, DH), BF),
            pltpu.VMEM((QB, HQ * DH), BF),
            pltpu.VMEM((NSTEPS, CHUNK, HALF), BF),
            pltpu.VMEM((NSTEPS, CHUNK, HALF), BF),
            pltpu.SemaphoreType.DMA((NSTEPS,)),
            pltpu.SemaphoreType.DMA((NSTEPS,)),
            pltpu.SemaphoreType.DMA((NSTEPS,)),
            pltpu.SemaphoreType.DMA((NSTEPS,)),
        ],
        compiler_params=pltpu.CompilerParams(
            collective_id=0, vmem_limit_bytes=100 * 1024 * 1024),
    )(x[0], wq_loc, K_ext[0], V_ext[0], wo_loc)
    return out[None]


# device time: 74089 ns/iter; 1.6368x vs baseline; 1.6368x over previous
import jax
import jax.numpy as jnp
from jax import lax
from jax.experimental import pallas as pl
from jax.experimental.pallas import tpu as pltpu

N_DEV = 4
SQ = 2048
D_MODEL = 1024
HQ = 8
DH = 128
QB = 256
WIN = 128
KW = 512
SCALE = 0.08838834764831843
CHUNK = SQ // N_DEV
HALF = D_MODEL // 2
NSTEPS = 2 * (N_DEV - 1)

BF = jnp.bfloat16


def _body(x_ref, wq_ref, k_ref, v_ref, wo_ref, out_ref,
          part_ref, kbf_ref, vbf_ref, ctx_ref,
          comm_r, comm_l, send_r, recv_r, send_l, recv_l):
    my = lax.axis_index("i")
    left = (my - 1) % N_DEV
    right = (my + 1) % N_DEV

    barrier_sem = pltpu.get_barrier_semaphore()
    for nbr in (left, right):
        pl.semaphore_signal(
            barrier_sem, inc=1,
            device_id=(nbr,), device_id_type=pl.DeviceIdType.MESH,
        )
    pl.semaphore_wait(barrier_sem, 2)

    kbf_ref[:, :] = k_ref[:, :].astype(BF)
    vbf_ref[:, :] = v_ref[:, :].astype(BF)

    def compute_chunk(c):
        for sub in range(CHUNK // QB):
            q0 = pl.multiple_of(c * CHUNK + sub * QB, QB)
            lo = pl.multiple_of(jnp.clip(q0 - WIN, 0, SQ - KW), WIN)
            x_blk = x_ref[pl.ds(q0, QB), :].astype(BF)
            q_all = (jnp.dot(x_blk, wq_ref[:, :],
                             preferred_element_type=jnp.float32)
                     * SCALE).astype(BF)
            for h in range(HQ):
                q_h = q_all[:, h * DH:(h + 1) * DH]
                k_h = kbf_ref[pl.ds(lo, KW), h * DH:(h + 1) * DH]
                v_h = vbf_ref[pl.ds(lo, KW), h * DH:(h + 1) * DH]
                s = lax.dot_general(q_h, k_h, (((1,), (1,)), ((), ())),
                                    preferred_element_type=jnp.float32)
                qi = q0 + lax.broadcasted_iota(jnp.int32, (QB, KW), 0)
                ki = lo + lax.broadcasted_iota(jnp.int32, (QB, KW), 1)
                s = jnp.where(jnp.abs(qi - ki) <= WIN, s, -1e9)
                m = jnp.max(s, axis=1, keepdims=True)
                w = jnp.exp(s - m)
                w = (w / jnp.sum(w, axis=1, keepdims=True)).astype(BF)
                ctx_ref[:, h * DH:(h + 1) * DH] = jnp.dot(
                    w, v_h, preferred_element_type=jnp.float32).astype(BF)
            part_ref[pl.ds(q0, QB), :] = jnp.dot(
                ctx_ref[:, :], wo_ref[:, :],
                preferred_element_type=jnp.float32).astype(BF)

    def rows_r(c):
        return part_ref.at[pl.ds(c * CHUNK, CHUNK), pl.ds(0, HALF)]

    def rows_l(c):
        return part_ref.at[pl.ds(c * CHUNK, CHUNK), pl.ds(HALF, HALF)]

    def acc_r(c, slot):
        part_ref[pl.ds(c * CHUNK, CHUNK), pl.ds(0, HALF)] = (
            part_ref[pl.ds(c * CHUNK, CHUNK), pl.ds(0, HALF)]
            + comm_r[slot])

    def acc_l(c, slot):
        part_ref[pl.ds(c * CHUNK, CHUNK), pl.ds(HALF, HALF)] = (
            part_ref[pl.ds(c * CHUNK, CHUNK), pl.ds(HALF, HALF)]
            + comm_l[slot])

    def rdma_r(src, slot, dev):
        return pltpu.make_async_remote_copy(
            src_ref=src, dst_ref=comm_r.at[slot],
            send_sem=send_r.at[slot], recv_sem=recv_r.at[slot],
            device_id=(dev,), device_id_type=pl.DeviceIdType.MESH)

    def rdma_l(src, slot, dev):
        return pltpu.make_async_remote_copy(
            src_ref=src, dst_ref=comm_l.at[slot],
            send_sem=send_l.at[slot], recv_sem=recv_l.at[slot],
            device_id=(dev,), device_id_type=pl.DeviceIdType.MESH)

    c0 = my
    c1 = (my - 1) % N_DEV
    c2 = (my + 1) % N_DEV
    c3 = (my + 2) % N_DEV

    compute_chunk(c0)
    r0 = rdma_r(rows_r(c0), 0, right); r0.start()
    l0 = rdma_l(rows_l(c0), 0, left); l0.start()

    compute_chunk(c1)
    r0.wait_recv()
    acc_r(c1, 0)
    r1 = rdma_r(rows_r(c1), 1, right); r1.start()

    compute_chunk(c2)
    l0.wait_recv()
    acc_l(c2, 0)
    l1 = rdma_l(rows_l(c2), 1, left); l1.start()

    compute_chunk(c3)
    r1.wait_recv()
    acc_r(c3, 1)
    r2 = rdma_r(rows_r(c3), 2, right); r2.start()
    l1.wait_recv()
    acc_l(c3, 1)
    l2 = rdma_l(rows_l(c3), 2, left); l2.start()

    r2.wait_recv()
    acc_r(c2, 2)
    ar0 = rdma_r(rows_r(c2), 3, right); ar0.start()
    l2.wait_recv()
    acc_l(c1, 2)
    al0 = rdma_l(rows_l(c1), 3, left); al0.start()

    out_ref[pl.ds(c2 * CHUNK, CHUNK), pl.ds(0, HALF)] = part_ref[
        pl.ds(c2 * CHUNK, CHUNK), pl.ds(0, HALF)].astype(jnp.float32)
    out_ref[pl.ds(c1 * CHUNK, CHUNK), pl.ds(HALF, HALF)] = part_ref[
        pl.ds(c1 * CHUNK, CHUNK), pl.ds(HALF, HALF)].astype(jnp.float32)

    ar0.wait_recv()
    ar1 = rdma_r(comm_r.at[3], 4, right); ar1.start()
    out_ref[pl.ds(c0 * CHUNK, CHUNK), pl.ds(0, HALF)] = (
        comm_r[3].astype(jnp.float32))
    al0.wait_recv()
    al1 = rdma_l(comm_l.at[3], 4, left); al1.start()
    out_ref[pl.ds(c0 * CHUNK, CHUNK), pl.ds(HALF, HALF)] = (
        comm_l[3].astype(jnp.float32))

    ar1.wait_recv()
    ar2 = rdma_r(comm_r.at[4], 5, right); ar2.start()
    out_ref[pl.ds(c1 * CHUNK, CHUNK), pl.ds(0, HALF)] = (
        comm_r[4].astype(jnp.float32))
    al1.wait_recv()
    al2 = rdma_l(comm_l.at[4], 5, left); al2.start()
    out_ref[pl.ds(c2 * CHUNK, CHUNK), pl.ds(HALF, HALF)] = (
        comm_l[4].astype(jnp.float32))

    ar2.wait_recv()
    out_ref[pl.ds(c3 * CHUNK, CHUNK), pl.ds(0, HALF)] = (
        comm_r[5].astype(jnp.float32))
    al2.wait_recv()
    out_ref[pl.ds(c3 * CHUNK, CHUNK), pl.ds(HALF, HALF)] = (
        comm_l[5].astype(jnp.float32))

    for r in (r0, r1, r2, l0, l1, l2, ar0, ar1, ar2, al0, al1, al2):
        r.wait_send()


def kernel(x, Wq, K_ext, V_ext, Wo):
    my = lax.axis_index("i")
    d_loc = HQ * DH
    wq_loc = lax.dynamic_slice(
        Wq, (0, my * d_loc), (Wq.shape[0], d_loc)).astype(BF)
    wo_loc = lax.dynamic_slice(
        Wo, (my * d_loc, 0), (d_loc, Wo.shape[1])).astype(BF)

    out = pl.pallas_call(
        _body,
        out_shape=jax.ShapeDtypeStruct((SQ, D_MODEL), jnp.float32),
        in_specs=[pl.BlockSpec(memory_space=pltpu.VMEM)] * 5,
        out_specs=pl.BlockSpec(memory_space=pltpu.VMEM),
        scratch_shapes=[
            pltpu.VMEM((SQ, D_MODEL), BF),
            pltpu.VMEM((SQ, HQ * DH), BF),
            pltpu.VMEM((SQ, HQ * DH), BF),
            pltpu.VMEM((QB, HQ * DH), BF),
            pltpu.VMEM((NSTEPS, CHUNK, HALF), BF),
            pltpu.VMEM((NSTEPS, CHUNK, HALF), BF),
            pltpu.SemaphoreType.DMA((NSTEPS,)),
            pltpu.SemaphoreType.DMA((NSTEPS,)),
            pltpu.SemaphoreType.DMA((NSTEPS,)),
            pltpu.SemaphoreType.DMA((NSTEPS,)),
        ],
        compiler_params=pltpu.CompilerParams(
            collective_id=0, vmem_limit_bytes=100 * 1024 * 1024),
    )(x[0], wq_loc, K_ext[0].reshape(SQ, HQ * DH),
      V_ext[0].reshape(SQ, HQ * DH), wo_loc)
    return out[None]
